# Initial kernel scaffold; baseline (speedup 1.0000x reference)
import jax
import jax.numpy as jnp
from jax.experimental import pallas as pl


def kernel(x, edge_src, edge_dst, edge_vec, W1, W2):
    raise NotImplementedError("write your pallas kernel here")


# speedup vs baseline: 1.5246x; 1.5246x over previous
import functools

import numpy as np
import jax
import jax.numpy as jnp
from jax import lax
from jax.experimental import pallas as pl
from jax.experimental.pallas import tpu as pltpu
from jax.experimental.pallas import tpu_sc as plsc

N_NODES = 100000
N_EDGES = 3200000
MAX_RADIUS = 3.0
NUM_BASIS = 10

G = 16384
TMAX = 11.0
CHUNK = 1024
KSUB = CHUNK // 128
NW = 32
NCHUNKS = N_EDGES // CHUNK

_CC = np.float32(1.14136 * np.exp(2.0))
_SCALE = np.float32(np.sqrt(2.0) / 4.0 / (np.sqrt(3.0) * 8.0))
_S3 = np.float32(np.sqrt(3.0))
_S3H = np.float32(np.sqrt(3.0) / 2.0)
_TSCALE = np.float32((G - 1) / 3.0)
_MAGIC = np.int32(0x5F3759DF)


def _table_body(w1_ref, w2_ref, out_ref):
    rows = lax.broadcasted_iota(jnp.int32, (G // 128, 128), 0)
    cols = lax.broadcasted_iota(jnp.int32, (G // 128, 128), 1)
    t = (rows * 128 + cols).astype(jnp.float32) * np.float32(TMAX / (G - 1))
    embs = []
    for k in range(1, NUM_BASIS + 1):
        d = t - np.float32(k)
        inb = (d > -1.0) & (d < 1.0)
        dd = jnp.where(inb, 1.0 - d * d, 1.0)
        embs.append(jnp.where(inb, _CC * jnp.exp(-2.0 / dd), 0.0))
    hs = []
    for j in range(16):
        acc = embs[0] * w1_ref[j]
        for k in range(1, NUM_BASIS):
            acc = acc + embs[k] * w1_ref[k * 16 + j]
        hs.append(jnp.maximum(acc, 0.0))
    for c in range(3):
        acc = hs[0] * w2_ref[c]
        for j in range(1, 16):
            acc = acc + hs[j] * w2_ref[j * 3 + c]
        out_ref[c] = acc * _SCALE
    out_ref[3] = jnp.zeros_like(t)


_tc_table = pl.pallas_call(
    _table_body,
    out_shape=jax.ShapeDtypeStruct((4, G // 128, 128), jnp.float32),
    in_specs=[pl.BlockSpec(memory_space=pltpu.SMEM),
              pl.BlockSpec(memory_space=pltpu.SMEM)],
)


def _sc_body(xpad_hbm, src2_hbm, vec1_hbm, tab_hbm, out_hbm,
             idx_v, vec_v, rows_v, tab_v, accv, sem):
    wid = lax.axis_index("s") * 2 + lax.axis_index("c")
    pltpu.sync_copy(tab_hbm, tab_v)
    iota = lax.iota(jnp.int32, 16)
    iota3 = iota * 3
    czero = jnp.zeros((16,), jnp.int32)
    cone = jnp.full((16,), 1, jnp.int32)
    ctwo = jnp.full((16,), 2, jnp.int32)
    nm = (NCHUNKS - wid + NW - 1) // NW

    def chunk_body(m, acc):
        cidx = wid + m * NW
        pltpu.sync_copy(src2_hbm.at[pl.ds(cidx * KSUB, KSUB)], idx_v)
        pltpu.sync_copy(vec1_hbm.at[pl.ds(cidx * (CHUNK * 3), CHUNK * 3)],
                        vec_v)
        cps = [pltpu.async_copy(xpad_hbm.at[idx_v.at[j]],
                                rows_v.at[pl.ds(j * 128, 128)], sem)
               for j in range(KSUB)]
        for cp in cps:
            cp.wait()

        def step(i, acc):
            b3 = i * 48
            vx = plsc.load_gather(vec_v, [iota3 + b3])
            vy = plsc.load_gather(vec_v, [iota3 + (b3 + 1)])
            vz = plsc.load_gather(vec_v, [iota3 + (b3 + 2)])
            r2 = jnp.maximum(vx * vx + vy * vy + vz * vz,
                             np.float32(1e-12))
            ib = _MAGIC - lax.shift_right_arithmetic(
                lax.bitcast_convert_type(r2, jnp.int32), 1)
            y = lax.bitcast_convert_type(ib, jnp.float32)
            y = y * (1.5 - 0.5 * r2 * y * y)
            y = y * (1.5 - 0.5 * r2 * y * y)
            y = y * (1.5 - 0.5 * r2 * y * y)
            nx = vx * y
            ny = vy * y
            nz = vz * y
            g = jnp.minimum(r2 * y * _TSCALE, np.float32(G - 1))
            i0 = jnp.minimum(g.astype(jnp.int32), G - 2)
            w = g - i0.astype(jnp.float32)
            t0a = plsc.load_gather(tab_v, [czero, i0])
            t1a = plsc.load_gather(tab_v, [cone, i0])
            t2a = plsc.load_gather(tab_v, [ctwo, i0])
            i1 = i0 + 1
            t0b = plsc.load_gather(tab_v, [czero, i1])
            t1b = plsc.load_gather(tab_v, [cone, i1])
            t2b = plsc.load_gather(tab_v, [ctwo, i1])
            tp0 = t0a + w * (t0b - t0a)
            tp1 = t1a + w * (t1b - t1a)
            tp2 = t2a + w * (t2b - t2a)
            ri = iota + i * 16
            x10 = plsc.load_gather(rows_v, [ri, czero])
            x11 = plsc.load_gather(rows_v, [ri, cone])
            x12 = plsc.load_gather(rows_v, [ri, ctwo])
            x13 = plsc.load_gather(rows_v, [ri, jnp.full((16,), 3, jnp.int32)])
            x14 = plsc.load_gather(rows_v, [ri, jnp.full((16,), 4, jnp.int32)])
            x15 = plsc.load_gather(rows_v, [ri, jnp.full((16,), 5, jnp.int32)])
            x16 = plsc.load_gather(rows_v, [ri, jnp.full((16,), 6, jnp.int32)])
            x17 = plsc.load_gather(rows_v, [ri, jnp.full((16,), 7, jnp.int32)])
            x18 = plsc.load_gather(rows_v, [ri, jnp.full((16,), 8, jnp.int32)])
            c1 = x11 * nx + x12 * ny + x13 * nz
            xx = nx * nx
            yy = ny * ny
            zz = nz * nz
            c2 = (_S3 * (nx * nz * x14 + nx * ny * x15 + ny * nz * x17)
                  + (yy - 0.5 * (xx + zz)) * x16 + _S3H * (zz - xx) * x18)
            return acc + (tp0 * x10 + tp1 * c1 + tp2 * c2)

        return lax.fori_loop(0, CHUNK // 16, step, acc)

    acc = lax.fori_loop(0, nm, chunk_body, jnp.zeros((16,), jnp.float32))
    accv[...] = acc
    pltpu.sync_copy(accv, out_hbm.at[wid])


_sc_main = functools.partial(
    pl.kernel,
    mesh=plsc.VectorSubcoreMesh(core_axis_name="c", subcore_axis_name="s"),
    out_type=jax.ShapeDtypeStruct((NW, 16), jnp.float32),
    scratch_types=[
        pltpu.VMEM((KSUB, 128), jnp.int32),
        pltpu.VMEM((CHUNK * 3,), jnp.float32),
        pltpu.VMEM((CHUNK, 16), jnp.float32),
        pltpu.VMEM((4, G), jnp.float32),
        pltpu.VMEM((16,), jnp.float32),
        pltpu.SemaphoreType.DMA,
    ],
    compiler_params=pltpu.CompilerParams(needs_layout_passes=False,
                                         use_tc_tiling_on_sc=False),
)(_sc_body)


def kernel(x, edge_src, edge_dst, edge_vec, W1, W2):
    tab = _tc_table(W1.reshape(-1), W2.reshape(-1)).reshape(4, G)
    xpad = jnp.pad(x, ((0, 0), (0, 7)))
    src2 = edge_src.reshape(N_EDGES // 128, 128)
    vec1 = edge_vec.reshape(N_EDGES * 3)
    parts = _sc_main(xpad, src2, vec1, tab)
    return jnp.sum(parts, dtype=jnp.float32).reshape(1)


# speedup vs baseline: 12.2396x; 8.0280x over previous
import functools

import numpy as np
import jax
import jax.numpy as jnp
from jax import lax
from jax.experimental import pallas as pl
from jax.experimental.pallas import tpu as pltpu
from jax.experimental.pallas import tpu_sc as plsc

N_NODES = 100000
N_EDGES = 3200000
MAX_RADIUS = 3.0
NUM_BASIS = 10

G = 16384
TMAX = 11.0
CHUNK = 1024
KSUB = CHUNK // 128
NW = 32
NCHUNKS = N_EDGES // CHUNK

_CC = np.float32(1.14136 * np.exp(2.0))
_SCALE = np.float32(np.sqrt(2.0) / 4.0 / (np.sqrt(3.0) * 8.0))
_S3 = np.float32(np.sqrt(3.0))
_S3H = np.float32(np.sqrt(3.0) / 2.0)
_TSCALE = np.float32((G - 1) / 3.0)
_MAGIC = np.int32(0x5F3759DF)


def _table_body(w1_ref, w2_ref, out_ref):
    rows = lax.broadcasted_iota(jnp.int32, (G // 128, 128), 0)
    cols = lax.broadcasted_iota(jnp.int32, (G // 128, 128), 1)
    t = (rows * 128 + cols).astype(jnp.float32) * np.float32(TMAX / (G - 1))
    embs = []
    for k in range(1, NUM_BASIS + 1):
        d = t - np.float32(k)
        inb = (d > -1.0) & (d < 1.0)
        dd = jnp.where(inb, 1.0 - d * d, 1.0)
        embs.append(jnp.where(inb, _CC * jnp.exp(-2.0 / dd), 0.0))
    hs = []
    for j in range(16):
        acc = embs[0] * w1_ref[j]
        for k in range(1, NUM_BASIS):
            acc = acc + embs[k] * w1_ref[k * 16 + j]
        hs.append(jnp.maximum(acc, 0.0))
    for c in range(3):
        acc = hs[0] * w2_ref[c]
        for j in range(1, 16):
            acc = acc + hs[j] * w2_ref[j * 3 + c]
        out_ref[c] = acc * _SCALE
    out_ref[3] = jnp.zeros_like(t)


_tc_table = pl.pallas_call(
    _table_body,
    out_shape=jax.ShapeDtypeStruct((4, G // 128, 128), jnp.float32),
    in_specs=[pl.BlockSpec(memory_space=pltpu.SMEM),
              pl.BlockSpec(memory_space=pltpu.SMEM)],
)


def _sc_body(xpad_hbm, src2_hbm, vecT_hbm, tab_hbm, out_hbm,
             idx_v, vx_v, vy_v, vz_v, rows_v, tab_v, accv, sem):
    wid = lax.axis_index("s") * 2 + lax.axis_index("c")
    pltpu.sync_copy(tab_hbm, tab_v)
    iota = lax.iota(jnp.int32, 16)
    czero = jnp.zeros((16,), jnp.int32)
    cone = jnp.full((16,), 1, jnp.int32)
    ctwo = jnp.full((16,), 2, jnp.int32)
    nm = (NCHUNKS - wid + NW - 1) // NW

    def chunk_body(m, acc):
        cidx = wid + m * NW
        base = cidx * CHUNK
        pltpu.sync_copy(src2_hbm.at[pl.ds(cidx * KSUB, KSUB)], idx_v)
        pltpu.sync_copy(vecT_hbm.at[0, pl.ds(base, CHUNK)], vx_v)
        pltpu.sync_copy(vecT_hbm.at[1, pl.ds(base, CHUNK)], vy_v)
        pltpu.sync_copy(vecT_hbm.at[2, pl.ds(base, CHUNK)], vz_v)
        cps = [pltpu.async_copy(xpad_hbm.at[idx_v.at[j]],
                                rows_v.at[pl.ds(j * 128, 128)], sem)
               for j in range(KSUB)]
        for cp in cps:
            cp.wait()

        def step(i, acc):
            vx = vx_v[pl.ds(i * 16, 16)]
            vy = vy_v[pl.ds(i * 16, 16)]
            vz = vz_v[pl.ds(i * 16, 16)]
            r2 = jnp.maximum(vx * vx + vy * vy + vz * vz,
                             np.float32(1e-12))
            ib = _MAGIC - lax.shift_right_arithmetic(
                lax.bitcast_convert_type(r2, jnp.int32), 1)
            y = lax.bitcast_convert_type(ib, jnp.float32)
            y = y * (1.5 - 0.5 * r2 * y * y)
            y = y * (1.5 - 0.5 * r2 * y * y)
            y = y * (1.5 - 0.5 * r2 * y * y)
            nx = vx * y
            ny = vy * y
            nz = vz * y
            g = jnp.minimum(r2 * y * _TSCALE, np.float32(G - 1))
            i0 = jnp.minimum(g.astype(jnp.int32), G - 2)
            w = g - i0.astype(jnp.float32)
            t0a = plsc.load_gather(tab_v, [czero, i0])
            t1a = plsc.load_gather(tab_v, [cone, i0])
            t2a = plsc.load_gather(tab_v, [ctwo, i0])
            i1 = i0 + 1
            t0b = plsc.load_gather(tab_v, [czero, i1])
            t1b = plsc.load_gather(tab_v, [cone, i1])
            t2b = plsc.load_gather(tab_v, [ctwo, i1])
            tp0 = t0a + w * (t0b - t0a)
            tp1 = t1a + w * (t1b - t1a)
            tp2 = t2a + w * (t2b - t2a)
            ri = iota + i * 16
            x10 = plsc.load_gather(rows_v, [ri, czero])
            x11 = plsc.load_gather(rows_v, [ri, cone])
            x12 = plsc.load_gather(rows_v, [ri, ctwo])
            x13 = plsc.load_gather(rows_v, [ri, jnp.full((16,), 3, jnp.int32)])
            x14 = plsc.load_gather(rows_v, [ri, jnp.full((16,), 4, jnp.int32)])
            x15 = plsc.load_gather(rows_v, [ri, jnp.full((16,), 5, jnp.int32)])
            x16 = plsc.load_gather(rows_v, [ri, jnp.full((16,), 6, jnp.int32)])
            x17 = plsc.load_gather(rows_v, [ri, jnp.full((16,), 7, jnp.int32)])
            x18 = plsc.load_gather(rows_v, [ri, jnp.full((16,), 8, jnp.int32)])
            c1 = x11 * nx + x12 * ny + x13 * nz
            xx = nx * nx
            yy = ny * ny
            zz = nz * nz
            c2 = (_S3 * (nx * nz * x14 + nx * ny * x15 + ny * nz * x17)
                  + (yy - 0.5 * (xx + zz)) * x16 + _S3H * (zz - xx) * x18)
            return acc + (tp0 * x10 + tp1 * c1 + tp2 * c2)

        return lax.fori_loop(0, CHUNK // 16, step, acc)

    acc = lax.fori_loop(0, nm, chunk_body, jnp.zeros((16,), jnp.float32))
    accv[...] = acc
    pltpu.sync_copy(accv, out_hbm.at[wid])


_sc_main = functools.partial(
    pl.kernel,
    mesh=plsc.VectorSubcoreMesh(core_axis_name="c", subcore_axis_name="s"),
    out_type=jax.ShapeDtypeStruct((NW, 16), jnp.float32),
    scratch_types=[
        pltpu.VMEM((KSUB, 128), jnp.int32),
        pltpu.VMEM((CHUNK,), jnp.float32),
        pltpu.VMEM((CHUNK,), jnp.float32),
        pltpu.VMEM((CHUNK,), jnp.float32),
        pltpu.VMEM((CHUNK, 16), jnp.float32),
        pltpu.VMEM((4, G), jnp.float32),
        pltpu.VMEM((16,), jnp.float32),
        pltpu.SemaphoreType.DMA,
    ],
    compiler_params=pltpu.CompilerParams(needs_layout_passes=False,
                                         use_tc_tiling_on_sc=False),
)(_sc_body)


def kernel(x, edge_src, edge_dst, edge_vec, W1, W2):
    tab = _tc_table(W1.reshape(-1), W2.reshape(-1)).reshape(4, G)
    xpad = jnp.pad(x, ((0, 0), (0, 7)))
    src2 = edge_src.reshape(N_EDGES // 128, 128)
    vecT = edge_vec.T
    parts = _sc_main(xpad, src2, vecT, tab)
    return jnp.sum(parts, dtype=jnp.float32).reshape(1)


# speedup vs baseline: 30.1645x; 2.4645x over previous
import functools

import numpy as np
import jax
import jax.numpy as jnp
from jax import lax
from jax.experimental import pallas as pl
from jax.experimental.pallas import tpu as pltpu
from jax.experimental.pallas import tpu_sc as plsc

N_NODES = 100000
N_EDGES = 3200000
MAX_RADIUS = 3.0
NUM_BASIS = 10

G = 16384
TMAX = 11.0
CHUNK = 1024
KSUB = CHUNK // 128
NW = 32
NM = 98
NCHUNKS = NW * NM
E_PAD = NCHUNKS * CHUNK

_CC = np.float32(1.14136 * np.exp(2.0))
_SCALE = np.float32(np.sqrt(2.0) / 4.0 / (np.sqrt(3.0) * 8.0))
_S3 = np.float32(np.sqrt(3.0))
_S3H = np.float32(np.sqrt(3.0) / 2.0)
_TSCALE = np.float32((G - 1) / 3.0)
_MAGIC = np.int32(0x5F3759DF)


def _table_body(w1_ref, w2_ref, out_ref):
    rows = lax.broadcasted_iota(jnp.int32, (G // 128, 128), 0)
    cols = lax.broadcasted_iota(jnp.int32, (G // 128, 128), 1)
    t = (rows * 128 + cols).astype(jnp.float32) * np.float32(TMAX / (G - 1))
    embs = []
    for k in range(1, NUM_BASIS + 1):
        d = t - np.float32(k)
        inb = (d > -1.0) & (d < 1.0)
        dd = jnp.where(inb, 1.0 - d * d, 1.0)
        embs.append(jnp.where(inb, _CC * jnp.exp(-2.0 / dd), 0.0))
    hs = []
    for j in range(16):
        acc = embs[0] * w1_ref[j]
        for k in range(1, NUM_BASIS):
            acc = acc + embs[k] * w1_ref[k * 16 + j]
        hs.append(jnp.maximum(acc, 0.0))
    for c in range(3):
        acc = hs[0] * w2_ref[c]
        for j in range(1, 16):
            acc = acc + hs[j] * w2_ref[j * 3 + c]
        out_ref[c] = acc * _SCALE
    out_ref[3] = jnp.zeros_like(t)


_tc_table = pl.pallas_call(
    _table_body,
    out_shape=jax.ShapeDtypeStruct((4, G // 128, 128), jnp.float32),
    in_specs=[pl.BlockSpec(memory_space=pltpu.SMEM),
              pl.BlockSpec(memory_space=pltpu.SMEM)],
)


def _sc_body(xpad_hbm, src2_hbm, vx_hbm, vy_hbm, vz_hbm, tab_hbm, out_hbm,
             idx0_v, idx1_v, vx0_v, vx1_v, vy0_v, vy1_v, vz0_v, vz1_v,
             rows0_v, rows1_v, tab_v, accv, semv0, semv1, semr0, semr1):
    wid = lax.axis_index("s") * 2 + lax.axis_index("c")
    pltpu.sync_copy(tab_hbm, tab_v)
    iota = lax.iota(jnp.int32, 16)
    czero = jnp.zeros((16,), jnp.int32)
    cone = jnp.full((16,), 1, jnp.int32)
    ctwo = jnp.full((16,), 2, jnp.int32)

    bufs = ((idx0_v, vx0_v, vy0_v, vz0_v, rows0_v, semv0, semr0),
            (idx1_v, vx1_v, vy1_v, vz1_v, rows1_v, semv1, semr1))

    def stage_copies(m, b):
        cidx = wid + m * NW
        base = cidx * CHUNK
        idx_v, vx_v, vy_v, vz_v, _, semv, _ = bufs[b]
        return (
            pltpu.make_async_copy(src2_hbm.at[pl.ds(cidx * KSUB, KSUB)],
                                  idx_v, semv),
            pltpu.make_async_copy(vx_hbm.at[pl.ds(base, CHUNK)], vx_v, semv),
            pltpu.make_async_copy(vy_hbm.at[pl.ds(base, CHUNK)], vy_v, semv),
            pltpu.make_async_copy(vz_hbm.at[pl.ds(base, CHUNK)], vz_v, semv),
        )

    def rows_copies(b):
        idx_v, _, _, _, rows_v, _, semr = bufs[b]
        return [pltpu.make_async_copy(xpad_hbm.at[idx_v.at[j]],
                                      rows_v.at[pl.ds(j * 128, 128)], semr)
                for j in range(KSUB)]

    for cp in stage_copies(0, 0):
        cp.start()
    for cp in stage_copies(0, 0):
        cp.wait()
    for cp in rows_copies(0):
        cp.start()

    def pair_body(jj, acc0):
        acc = acc0
        for q in (0, 1):
            m = jj * 2 + q
            nb = 1 - q
            _, vx_v, vy_v, vz_v, rows_v, _, _ = bufs[q]

            @pl.when(m + 1 < NM)
            def _():
                for cp in stage_copies(m + 1, nb):
                    cp.start()
                for cp in stage_copies(m + 1, nb):
                    cp.wait()
                for cp in rows_copies(nb):
                    cp.start()

            for cp in rows_copies(q):
                cp.wait()

            def step(i, acc):
                vx = vx_v[pl.ds(i * 16, 16)]
                vy = vy_v[pl.ds(i * 16, 16)]
                vz = vz_v[pl.ds(i * 16, 16)]
                r2 = jnp.maximum(vx * vx + vy * vy + vz * vz,
                                 np.float32(1e-12))
                ib = _MAGIC - lax.shift_right_arithmetic(
                    lax.bitcast_convert_type(r2, jnp.int32), 1)
                y = lax.bitcast_convert_type(ib, jnp.float32)
                y = y * (1.5 - 0.5 * r2 * y * y)
                y = y * (1.5 - 0.5 * r2 * y * y)
                y = y * (1.5 - 0.5 * r2 * y * y)
                nx = vx * y
                ny = vy * y
                nz = vz * y
                g = jnp.minimum(r2 * y * _TSCALE, np.float32(G - 1))
                i0 = jnp.minimum(g.astype(jnp.int32), G - 2)
                w = g - i0.astype(jnp.float32)
                t0a = plsc.load_gather(tab_v, [czero, i0])
                t1a = plsc.load_gather(tab_v, [cone, i0])
                t2a = plsc.load_gather(tab_v, [ctwo, i0])
                i1 = i0 + 1
                t0b = plsc.load_gather(tab_v, [czero, i1])
                t1b = plsc.load_gather(tab_v, [cone, i1])
                t2b = plsc.load_gather(tab_v, [ctwo, i1])
                tp0 = t0a + w * (t0b - t0a)
                tp1 = t1a + w * (t1b - t1a)
                tp2 = t2a + w * (t2b - t2a)
                ri = iota + i * 16
                x10 = plsc.load_gather(rows_v, [ri, czero])
                x11 = plsc.load_gather(rows_v, [ri, cone])
                x12 = plsc.load_gather(rows_v, [ri, ctwo])
                x13 = plsc.load_gather(rows_v, [ri, jnp.full((16,), 3, jnp.int32)])
                x14 = plsc.load_gather(rows_v, [ri, jnp.full((16,), 4, jnp.int32)])
                x15 = plsc.load_gather(rows_v, [ri, jnp.full((16,), 5, jnp.int32)])
                x16 = plsc.load_gather(rows_v, [ri, jnp.full((16,), 6, jnp.int32)])
                x17 = plsc.load_gather(rows_v, [ri, jnp.full((16,), 7, jnp.int32)])
                x18 = plsc.load_gather(rows_v, [ri, jnp.full((16,), 8, jnp.int32)])
                c1 = x11 * nx + x12 * ny + x13 * nz
                xx = nx * nx
                yy = ny * ny
                zz = nz * nz
                c2 = (_S3 * (nx * nz * x14 + nx * ny * x15 + ny * nz * x17)
                      + (yy - 0.5 * (xx + zz)) * x16 + _S3H * (zz - xx) * x18)
                return acc + (tp0 * x10 + tp1 * c1 + tp2 * c2)

            acc = lax.fori_loop(0, CHUNK // 16, step, acc)
        return acc

    acc = lax.fori_loop(0, NM // 2, pair_body, jnp.zeros((16,), jnp.float32))
    accv[...] = acc
    pltpu.sync_copy(accv, out_hbm.at[wid])


_sc_main = functools.partial(
    pl.kernel,
    mesh=plsc.VectorSubcoreMesh(core_axis_name="c", subcore_axis_name="s"),
    out_type=jax.ShapeDtypeStruct((NW, 16), jnp.float32),
    scratch_types=[
        pltpu.VMEM((KSUB, 128), jnp.int32),
        pltpu.VMEM((KSUB, 128), jnp.int32),
        pltpu.VMEM((CHUNK,), jnp.float32),
        pltpu.VMEM((CHUNK,), jnp.float32),
        pltpu.VMEM((CHUNK,), jnp.float32),
        pltpu.VMEM((CHUNK,), jnp.float32),
        pltpu.VMEM((CHUNK,), jnp.float32),
        pltpu.VMEM((CHUNK,), jnp.float32),
        pltpu.VMEM((CHUNK, 16), jnp.float32),
        pltpu.VMEM((CHUNK, 16), jnp.float32),
        pltpu.VMEM((4, G), jnp.float32),
        pltpu.VMEM((16,), jnp.float32),
        pltpu.SemaphoreType.DMA,
        pltpu.SemaphoreType.DMA,
        pltpu.SemaphoreType.DMA,
        pltpu.SemaphoreType.DMA,
    ],
    compiler_params=pltpu.CompilerParams(needs_layout_passes=False,
                                         use_tc_tiling_on_sc=False),
)(_sc_body)


def kernel(x, edge_src, edge_dst, edge_vec, W1, W2):
    tab = _tc_table(W1.reshape(-1), W2.reshape(-1)).reshape(4, G)
    xpad = jnp.pad(x, ((0, 0), (0, 7)))
    pad_e = E_PAD - N_EDGES
    src2 = jnp.pad(edge_src, (0, pad_e)).reshape(E_PAD // 128, 128)
    parts = _sc_main(xpad, src2,
                     jnp.pad(edge_vec[:, 0], (0, pad_e)),
                     jnp.pad(edge_vec[:, 1], (0, pad_e)),
                     jnp.pad(edge_vec[:, 2], (0, pad_e)), tab)
    return jnp.sum(parts, dtype=jnp.float32).reshape(1)
